# baseline (speedup 1.0000x reference)
import functools

import jax
import jax.numpy as jnp
from jax import lax
from jax.experimental import pallas as pl
from jax.experimental.pallas import tpu as pltpu
from jax.experimental.pallas import tpu_sc as plsc

D_MODEL = 1024
B_TOTAL = 4 * 4096
B_SC = 8192
B_TC = B_TOTAL - B_SC
NUM_CORES = 2
NUM_SUBCORES = 16
NW = NUM_CORES * NUM_SUBCORES
B_PER_W = B_SC // NW
CHUNK = 32
NCHUNK = B_PER_W // CHUNK
NBUF = 3
RB = 1024
NB_TC = B_TC // RB


def _pe_gather(x_grouped, pe):
    mesh = plsc.VectorSubcoreMesh(core_axis_name="c", subcore_axis_name="s")

    @functools.partial(
        pl.kernel,
        mesh=mesh,
        out_type=jax.ShapeDtypeStruct((B_TOTAL, D_MODEL), jnp.float32),
        scratch_types=[
            pltpu.VMEM((NCHUNK, CHUNK), jnp.int32),
        ]
        + [pltpu.VMEM((CHUNK, D_MODEL), jnp.float32) for _ in range(NBUF)]
        + [pltpu.SemaphoreType.DMA for _ in range(2 * NBUF)],
    )
    def k(idx_hbm, table_hbm, out_hbm, idx_v, *scratch):
        bufs = scratch[:NBUF]
        gsems = scratch[NBUF:2 * NBUF]
        osems = scratch[2 * NBUF:]
        wid = lax.axis_index("s") * NUM_CORES + lax.axis_index("c")
        base = wid * B_PER_W
        pltpu.sync_copy(idx_hbm.at[wid], idx_v)
        gcp = [None] * NBUF
        ocp = [None] * NBUF
        for g in range(min(NBUF, NCHUNK)):
            gcp[g] = pltpu.async_copy(
                table_hbm.at[idx_v.at[g]], bufs[g], gsems[g])
        for c in range(NCHUNK):
            b = c % NBUF
            gcp[b].wait()
            ocp[b] = pltpu.async_copy(
                bufs[b], out_hbm.at[pl.ds(base + c * CHUNK, CHUNK)],
                osems[b])
            g = c + NBUF
            if g < NCHUNK:
                ocp[b].wait()
                gcp[b] = pltpu.async_copy(
                    table_hbm.at[idx_v.at[g]], bufs[b], gsems[b])
        for c in range(max(0, NCHUNK - NBUF), NCHUNK):
            ocp[c % NBUF].wait()

    return k(x_grouped, pe)


def _tc_body(x_ref, s1_ref, c1_ref, s2_ref, c2_ref, out_ref):
    xb = x_ref[0, 0, :].reshape(RB, 1)
    ia = lax.broadcasted_iota(jnp.int32, (RB, 128), 1)
    ib = lax.broadcasted_iota(jnp.int32, (RB, 64), 1)
    onea = ((xb >> 6) == ia).astype(jnp.bfloat16)
    oneb = ((xb & 63) == ib).astype(jnp.bfloat16)
    sa = jnp.dot(onea, s1_ref[...], preferred_element_type=jnp.float32)
    ca = jnp.dot(onea, c1_ref[...], preferred_element_type=jnp.float32)
    sb = jnp.dot(oneb, s2_ref[...], preferred_element_type=jnp.float32)
    cb = jnp.dot(oneb, c2_ref[...], preferred_element_type=jnp.float32)
    out_ref[...] = sa * cb + ca * sb


def _pe_compute(x2, s1, c1, s2, c2):
    return pl.pallas_call(
        _tc_body,
        grid=(NB_TC,),
        in_specs=[pl.BlockSpec((1, 1, RB), lambda i: (i, 0, 0))]
        + [pl.BlockSpec(t, lambda i: (0, 0))
           for t in ((128, D_MODEL), (128, D_MODEL),
                     (64, D_MODEL), (64, D_MODEL))],
        out_specs=pl.BlockSpec((RB, D_MODEL), lambda i: (i, 0)),
        out_shape=jax.ShapeDtypeStruct((B_TC, D_MODEL), jnp.float32),
    )(x2, s1, c1, s2, c2)


def kernel(x, pe):
    x_flat = x.reshape(-1).astype(jnp.int32)
    x_sc = x_flat[:B_SC].reshape(NW, NCHUNK, CHUNK)
    x_tc = x_flat[B_SC:].reshape(NB_TC, 1, RB)
    div_term = jnp.exp(
        jnp.arange(0, D_MODEL, 2, dtype=jnp.float32)
        * -(jnp.log(jnp.float32(10000.0)) / D_MODEL))
    wfull = jnp.repeat(div_term, 2)
    phase = jnp.tile(jnp.array([0.0, jnp.pi / 2], dtype=jnp.float32),
                     D_MODEL // 2)
    ang_a = (jnp.arange(128, dtype=jnp.float32) * 64.0)[:, None] * wfull
    ang_b = jnp.arange(64, dtype=jnp.float32)[:, None] * wfull + phase
    s1 = jnp.sin(ang_a).astype(jnp.bfloat16)
    c1 = jnp.cos(ang_a).astype(jnp.bfloat16)
    s2 = jnp.sin(ang_b).astype(jnp.bfloat16)
    c2 = jnp.cos(ang_b).astype(jnp.bfloat16)
    out_sc = _pe_gather(x_sc, pe.astype(jnp.float32))
    out_tc = _pe_compute(x_tc, s1, c1, s2, c2)
    out = lax.dynamic_update_slice(out_sc, out_tc, (B_SC, 0))
    return out.reshape(x.shape + (D_MODEL,))


# speedup vs baseline: 1.1705x; 1.1705x over previous
import functools

import jax
import jax.numpy as jnp
from jax import lax
from jax.experimental import pallas as pl
from jax.experimental.pallas import tpu as pltpu
from jax.experimental.pallas import tpu_sc as plsc

D_MODEL = 1024
BATCH = 4
SEQ = 4096
B_TOTAL = BATCH * SEQ
NUM_CORES = 2
NUM_SUBCORES = 16
NW = NUM_CORES * NUM_SUBCORES
B_PER_W = B_TOTAL // NW
W_PER_BATCH = NW // BATCH
CHUNK = 32
NCHUNK = B_PER_W // CHUNK
NBUF = 3


def _pe_gather(x, pe):
    mesh = plsc.VectorSubcoreMesh(core_axis_name="c", subcore_axis_name="s")

    @functools.partial(
        pl.kernel,
        mesh=mesh,
        out_type=jax.ShapeDtypeStruct((B_TOTAL, D_MODEL), jnp.float32),
        scratch_types=[
            pltpu.VMEM((B_PER_W,), jnp.int32),
        ]
        + [pltpu.VMEM((CHUNK, D_MODEL), jnp.float32) for _ in range(NBUF)]
        + [pltpu.SemaphoreType.DMA for _ in range(2 * NBUF)],
    )
    def k(idx_hbm, table_hbm, out_hbm, idx_v, *scratch):
        bufs = scratch[:NBUF]
        gsems = scratch[NBUF:2 * NBUF]
        osems = scratch[2 * NBUF:]
        wid = lax.axis_index("s") * NUM_CORES + lax.axis_index("c")
        batch = wid // W_PER_BATCH
        col0 = (wid % W_PER_BATCH) * B_PER_W
        base = wid * B_PER_W
        pltpu.sync_copy(idx_hbm.at[batch, pl.ds(col0, B_PER_W)], idx_v)
        gcp = [None] * NBUF
        ocp = [None] * NBUF
        for g in range(NBUF):
            gcp[g] = pltpu.async_copy(
                table_hbm.at[idx_v.at[pl.ds(g * CHUNK, CHUNK)]],
                bufs[g], gsems[g])
        for c in range(NCHUNK):
            b = c % NBUF
            gcp[b].wait()
            ocp[b] = pltpu.async_copy(
                bufs[b], out_hbm.at[pl.ds(base + c * CHUNK, CHUNK)],
                osems[b])
            g = c + NBUF
            if g < NCHUNK:
                ocp[b].wait()
                gcp[b] = pltpu.async_copy(
                    table_hbm.at[idx_v.at[pl.ds(g * CHUNK, CHUNK)]],
                    bufs[b], gsems[b])
        for c in range(NCHUNK - NBUF, NCHUNK):
            ocp[c % NBUF].wait()

    return k(x, pe)


def kernel(x, pe):
    if x.dtype != jnp.int32:
        x = x.astype(jnp.int32)
    if pe.dtype != jnp.float32:
        pe = pe.astype(jnp.float32)
    out = _pe_gather(x, pe)
    return out.reshape(x.shape + (D_MODEL,))


# speedup vs baseline: 1.1777x; 1.0062x over previous
import functools

import jax
import jax.numpy as jnp
from jax import lax
from jax.experimental import pallas as pl
from jax.experimental.pallas import tpu as pltpu
from jax.experimental.pallas import tpu_sc as plsc

D_MODEL = 1024
BATCH = 4
SEQ = 4096
B_TOTAL = BATCH * SEQ
NUM_CORES = 2
NUM_SUBCORES = 16
NW = NUM_CORES * NUM_SUBCORES
B_PER_W = B_TOTAL // NW
W_PER_BATCH = NW // BATCH
CHUNK = 16
NCHUNK = B_PER_W // CHUNK
NBUF = 6


def _pe_gather(x, pe):
    mesh = plsc.VectorSubcoreMesh(core_axis_name="c", subcore_axis_name="s")

    @functools.partial(
        pl.kernel,
        mesh=mesh,
        out_type=jax.ShapeDtypeStruct((B_TOTAL, D_MODEL), jnp.float32),
        scratch_types=[
            pltpu.VMEM((B_PER_W,), jnp.int32),
        ]
        + [pltpu.VMEM((CHUNK, D_MODEL), jnp.float32) for _ in range(NBUF)]
        + [pltpu.SemaphoreType.DMA for _ in range(2 * NBUF)],
    )
    def k(idx_hbm, table_hbm, out_hbm, idx_v, *scratch):
        bufs = scratch[:NBUF]
        gsems = scratch[NBUF:2 * NBUF]
        osems = scratch[2 * NBUF:]
        wid = lax.axis_index("s") * NUM_CORES + lax.axis_index("c")
        batch = wid // W_PER_BATCH
        col0 = (wid % W_PER_BATCH) * B_PER_W
        base = wid * B_PER_W
        pltpu.sync_copy(idx_hbm.at[batch, pl.ds(col0, B_PER_W)], idx_v)
        gcp = [None] * NBUF
        ocp = [None] * NBUF
        for g in range(NBUF):
            gcp[g] = pltpu.async_copy(
                table_hbm.at[idx_v.at[pl.ds(g * CHUNK, CHUNK)]],
                bufs[g], gsems[g])
        for c in range(NCHUNK):
            b = c % NBUF
            gcp[b].wait()
            ocp[b] = pltpu.async_copy(
                bufs[b], out_hbm.at[pl.ds(base + c * CHUNK, CHUNK)],
                osems[b])
            g = c + NBUF
            if g < NCHUNK:
                ocp[b].wait()
                gcp[b] = pltpu.async_copy(
                    table_hbm.at[idx_v.at[pl.ds(g * CHUNK, CHUNK)]],
                    bufs[b], gsems[b])
        for c in range(NCHUNK - NBUF, NCHUNK):
            ocp[c % NBUF].wait()

    return k(x, pe)


def kernel(x, pe):
    if x.dtype != jnp.int32:
        x = x.astype(jnp.int32)
    if pe.dtype != jnp.float32:
        pe = pe.astype(jnp.float32)
    out = _pe_gather(x, pe)
    return out.reshape(x.shape + (D_MODEL,))
